# Initial kernel scaffold; baseline (speedup 1.0000x reference)
import jax
import jax.numpy as jnp
from jax.experimental import pallas as pl


def kernel(x, row_src, row_dst, row_val, col_src, col_dst, col_val, pre_W, pre_b, bnp_g, bnp_b, ln0_g, ln0_b, Wr0, br0, Wc0, bc0, bnl0_g, bnl0_b, ln1_g, ln1_b, Wr1, br1, Wc1, bc1, bnl1_g, bnl1_b, out_W, out_b):
    raise NotImplementedError("write your pallas kernel here")


# speedup vs baseline: 5.3272x; 5.3272x over previous
import functools
import math

import jax
import jax.numpy as jnp
from jax import lax
from jax.experimental import pallas as pl
from jax.experimental.pallas import tpu as pltpu
from jax.experimental.pallas import tpu_sc as plsc

N = 50176
C = 200
HID = 64
NCLS = 16
DEG = 16
E = N * DEG

FH = HID // 2
NSUB = 16
EPT = E // NSUB
BLK = 512
NBLK = EPT // BLK
CH = 128
NCH = BLK // CH
RPT = N // NSUB
_BN_S = 1.0 / math.sqrt(1.0 + 1e-5)
_R = 512
_G = N // _R


def _ln(h, g, b):
    m = jnp.mean(h, axis=1, keepdims=True)
    v = jnp.mean((h - m) ** 2, axis=1, keepdims=True)
    return (h - m) * lax.rsqrt(v + 1e-5) * g + b


def _split(z):
    return jnp.stack([z[:, :FH], z[:, FH:]])


def _dense_pre_body(x_ref, pw, pb, bg, bb, lg, lb, wr, wbr, wc, wbc,
                    h_out, rv_out, cv_out):
    x = x_ref[...]
    t = jnp.dot(x, pw[...], preferred_element_type=jnp.float32) + pb[...]
    h = t * (_BN_S * bg[...]) + bb[...]
    hn = _ln(h, lg[...], lb[...])
    h_out[...] = hn
    rv = jnp.dot(hn, wr[...], preferred_element_type=jnp.float32) + wbr[...]
    cv = jnp.dot(hn, wc[...], preferred_element_type=jnp.float32) + wbc[...]
    rv_out[...] = _split(rv)
    cv_out[...] = _split(cv)


def _dense_mid_body(acc_ref, h_ref, bg, bb, lg, lb, wr, wbr, wc, wbc,
                    h_out, rv_out, cv_out):
    acc = acc_ref[...]
    a = jnp.concatenate([acc[0], acc[1]], axis=1) + h_ref[...]
    hb = a * (_BN_S * bg[...]) + bb[...]
    h = jnp.where(hb > 0, hb, 0.01 * hb)
    hn = _ln(h, lg[...], lb[...])
    h_out[...] = hn
    rv = jnp.dot(hn, wr[...], preferred_element_type=jnp.float32) + wbr[...]
    cv = jnp.dot(hn, wc[...], preferred_element_type=jnp.float32) + wbc[...]
    rv_out[...] = _split(rv)
    cv_out[...] = _split(cv)


def _dense_out_body(acc_ref, h_ref, bg, bb, ow, ob, p_out):
    acc = acc_ref[...]
    a = jnp.concatenate([acc[0], acc[1]], axis=1) + h_ref[...]
    hb = a * (_BN_S * bg[...]) + bb[...]
    h = jnp.where(hb > 0, hb, 0.01 * hb)
    logits = jnp.dot(h, ow[...], preferred_element_type=jnp.float32) + ob[...]
    mx = jnp.max(logits, axis=1, keepdims=True)
    ex = jnp.exp(logits - mx)
    p_out[...] = ex / jnp.sum(ex, axis=1, keepdims=True)


def _vec_spec():
    return pl.BlockSpec((1, HID), lambda i: (0, 0))


def _dense_pre(x, pre_W, pre_b, bnp_g, bnp_b, ln_g, ln_b, Wr, br, Wc, bc):
    return pl.pallas_call(
        _dense_pre_body,
        grid=(_G,),
        in_specs=[
            pl.BlockSpec((_R, C), lambda i: (i, 0)),
            pl.BlockSpec((C, HID), lambda i: (0, 0)),
            _vec_spec(), _vec_spec(), _vec_spec(), _vec_spec(), _vec_spec(),
            pl.BlockSpec((HID, HID), lambda i: (0, 0)),
            _vec_spec(),
            pl.BlockSpec((HID, HID), lambda i: (0, 0)),
            _vec_spec(),
        ],
        out_specs=[
            pl.BlockSpec((_R, HID), lambda i: (i, 0)),
            pl.BlockSpec((2, _R, FH), lambda i: (0, i, 0)),
            pl.BlockSpec((2, _R, FH), lambda i: (0, i, 0)),
        ],
        out_shape=[
            jax.ShapeDtypeStruct((N, HID), jnp.float32),
            jax.ShapeDtypeStruct((2, N, FH), jnp.float32),
            jax.ShapeDtypeStruct((2, N, FH), jnp.float32),
        ],
    )(x, pre_W, pre_b.reshape(1, -1), bnp_g.reshape(1, -1),
      bnp_b.reshape(1, -1), ln_g.reshape(1, -1), ln_b.reshape(1, -1),
      Wr, br.reshape(1, -1), Wc, bc.reshape(1, -1))


def _dense_mid(acc, h, bnl_g, bnl_b, ln_g, ln_b, Wr, br, Wc, bc):
    return pl.pallas_call(
        _dense_mid_body,
        grid=(_G,),
        in_specs=[
            pl.BlockSpec((2, _R, FH), lambda i: (0, i, 0)),
            pl.BlockSpec((_R, HID), lambda i: (i, 0)),
            _vec_spec(), _vec_spec(), _vec_spec(), _vec_spec(),
            pl.BlockSpec((HID, HID), lambda i: (0, 0)),
            _vec_spec(),
            pl.BlockSpec((HID, HID), lambda i: (0, 0)),
            _vec_spec(),
        ],
        out_specs=[
            pl.BlockSpec((_R, HID), lambda i: (i, 0)),
            pl.BlockSpec((2, _R, FH), lambda i: (0, i, 0)),
            pl.BlockSpec((2, _R, FH), lambda i: (0, i, 0)),
        ],
        out_shape=[
            jax.ShapeDtypeStruct((N, HID), jnp.float32),
            jax.ShapeDtypeStruct((2, N, FH), jnp.float32),
            jax.ShapeDtypeStruct((2, N, FH), jnp.float32),
        ],
    )(acc, h, bnl_g.reshape(1, -1), bnl_b.reshape(1, -1),
      ln_g.reshape(1, -1), ln_b.reshape(1, -1), Wr, br.reshape(1, -1),
      Wc, bc.reshape(1, -1))


def _dense_out(acc, h, bnl_g, bnl_b, out_W, out_b):
    return pl.pallas_call(
        _dense_out_body,
        grid=(_G,),
        in_specs=[
            pl.BlockSpec((2, _R, FH), lambda i: (0, i, 0)),
            pl.BlockSpec((_R, HID), lambda i: (i, 0)),
            _vec_spec(), _vec_spec(),
            pl.BlockSpec((HID, NCLS), lambda i: (0, 0)),
            pl.BlockSpec((1, NCLS), lambda i: (0, 0)),
        ],
        out_specs=[pl.BlockSpec((_R, NCLS), lambda i: (i, 0))],
        out_shape=[jax.ShapeDtypeStruct((N, NCLS), jnp.float32)],
    )(acc, h, bnl_g.reshape(1, -1), bnl_b.reshape(1, -1),
      out_W, out_b.reshape(1, -1))[0]


def _sc_body(rowv, colv, rs, rd, rvals, cs, cd, cvals, out,
             accum, idxb, srcb, valb, gbuf):
    c = lax.axis_index("c")
    s = lax.axis_index("s")
    coff = c * N
    r0 = s * RPT

    def _zb(e, _):
        z = jnp.zeros((16,), jnp.float32)
        gbuf[e, pl.ds(0, 16)] = z
        gbuf[e, pl.ds(16, 16)] = z
        return 0
    lax.fori_loop(0, BLK, _zb, 0)
    for k in range(RPT // BLK):
        pltpu.sync_copy(gbuf, accum.at[pl.ds(r0 + k * BLK, BLK)])
    rem = RPT % BLK
    if rem:
        pltpu.sync_copy(gbuf.at[pl.ds(0, rem)],
                        accum.at[pl.ds(r0 + (RPT // BLK) * BLK, rem)])
    plsc.subcore_barrier()

    def run_edges(src2, dst2, vals, feat):
        ebase2 = s * (EPT // CH)

        def blk_body(b, _):
            o2 = ebase2 + b * NCH
            pltpu.sync_copy(dst2.at[pl.ds(o2, NCH)], idxb)
            pltpu.sync_copy(src2.at[pl.ds(o2, NCH)], srcb)
            pltpu.sync_copy(vals.at[pl.ds(s * EPT + b * BLK, BLK)], valb)

            def _adj(i, _):
                k = i // (CH // 16)
                j = (i % (CH // 16)) * 16
                idxb[k, pl.ds(j, 16)] = idxb[k, pl.ds(j, 16)] + coff
                return 0
            lax.fori_loop(0, BLK // 16, _adj, 0)

            for k in range(NCH):
                pltpu.sync_copy(feat.at[idxb.at[k]],
                                gbuf.at[pl.ds(k * CH, CH)])

            def _mul(g, _):
                vv = valb[pl.ds(g * 16, 16)]
                for j in range(16):
                    e = g * 16 + j
                    v = vv[j]
                    gbuf[e, pl.ds(0, 16)] = gbuf[e, pl.ds(0, 16)] * v
                    gbuf[e, pl.ds(16, 16)] = gbuf[e, pl.ds(16, 16)] * v
                return 0
            lax.fori_loop(0, BLK // 16, _mul, 0)

            for k in range(NCH):
                pltpu.sync_copy(gbuf.at[pl.ds(k * CH, CH)],
                                accum.at[srcb.at[k]], add=True)
            return 0

        lax.fori_loop(0, NBLK, blk_body, 0)

    run_edges(rs, rd, rvals, rowv)
    run_edges(cs, cd, cvals, colv)
    plsc.subcore_barrier()

    for k in range(RPT // BLK):
        pltpu.sync_copy(accum.at[pl.ds(r0 + k * BLK, BLK)],
                        out.at[pl.ds(coff + r0 + k * BLK, BLK)])
    if rem:
        pltpu.sync_copy(accum.at[pl.ds(r0 + (RPT // BLK) * BLK, rem)],
                        out.at[pl.ds(coff + r0 + (RPT // BLK) * BLK, rem)])


_sc_edge = pl.kernel(
    _sc_body,
    out_type=jax.ShapeDtypeStruct((2 * N, FH), jnp.float32),
    mesh=plsc.VectorSubcoreMesh(core_axis_name="c", subcore_axis_name="s"),
    compiler_params=pltpu.CompilerParams(use_tc_tiling_on_sc=False),
    scratch_types=[
        pltpu.VMEM_SHARED((N, FH), jnp.float32),
        pltpu.VMEM((NCH, CH), jnp.int32),
        pltpu.VMEM((NCH, CH), jnp.int32),
        pltpu.VMEM((BLK,), jnp.float32),
        pltpu.VMEM((BLK, FH), jnp.float32),
    ],
)


def kernel(x, row_src, row_dst, row_val, col_src, col_dst, col_val,
           pre_W, pre_b, bnp_g, bnp_b,
           ln0_g, ln0_b, Wr0, br0, Wc0, bc0, bnl0_g, bnl0_b,
           ln1_g, ln1_b, Wr1, br1, Wc1, bc1, bnl1_g, bnl1_b,
           out_W, out_b):
    rs2 = row_src.reshape(-1, CH)
    rd2 = row_dst.reshape(-1, CH)
    cs2 = col_src.reshape(-1, CH)
    cd2 = col_dst.reshape(-1, CH)

    h0, rv0, cv0 = _dense_pre(x, pre_W, pre_b, bnp_g, bnp_b,
                              ln0_g, ln0_b, Wr0, br0, Wc0, bc0)
    acc0 = _sc_edge(rv0.reshape(2 * N, FH), cv0.reshape(2 * N, FH),
                    rs2, rd2, row_val, cs2, cd2, col_val)
    h1, rv1, cv1 = _dense_mid(acc0.reshape(2, N, FH), h0, bnl0_g, bnl0_b,
                              ln1_g, ln1_b, Wr1, br1, Wc1, bc1)
    acc1 = _sc_edge(rv1.reshape(2 * N, FH), cv1.reshape(2 * N, FH),
                    rs2, rd2, row_val, cs2, cd2, col_val)
    return _dense_out(acc1.reshape(2, N, FH), h1, bnl1_g, bnl1_b,
                      out_W, out_b)


# speedup vs baseline: 11.3846x; 2.1371x over previous
import functools
import math

import jax
import jax.numpy as jnp
from jax import lax
from jax.experimental import pallas as pl
from jax.experimental.pallas import tpu as pltpu
from jax.experimental.pallas import tpu_sc as plsc

N = 50176
C = 200
HID = 64
NCLS = 16
DEG = 16
E = N * DEG

FH = HID // 2
NSUB = 16
EPT = E // NSUB
BLK = 512
NBLK = EPT // BLK
CH = 128
NCH = BLK // CH
RPT = N // NSUB
_BN_S = 1.0 / math.sqrt(1.0 + 1e-5)
_R = 512
_G = N // _R


def _ln(h, g, b):
    m = jnp.mean(h, axis=1, keepdims=True)
    v = jnp.mean((h - m) ** 2, axis=1, keepdims=True)
    return (h - m) * lax.rsqrt(v + 1e-5) * g + b


def _split(z):
    return jnp.stack([z[:, :FH], z[:, FH:]])


def _dense_pre_body(x_ref, pw, pb, bg, bb, lg, lb, wr, wbr, wc, wbc,
                    h_out, rv_out, cv_out):
    x = x_ref[...]
    t = jnp.dot(x, pw[...], preferred_element_type=jnp.float32) + pb[...]
    h = t * (_BN_S * bg[...]) + bb[...]
    hn = _ln(h, lg[...], lb[...])
    h_out[...] = hn
    rv = jnp.dot(hn, wr[...], preferred_element_type=jnp.float32) + wbr[...]
    cv = jnp.dot(hn, wc[...], preferred_element_type=jnp.float32) + wbc[...]
    rv_out[...] = _split(rv)
    cv_out[...] = _split(cv)


def _dense_mid_body(acc_ref, h_ref, bg, bb, lg, lb, wr, wbr, wc, wbc,
                    h_out, rv_out, cv_out):
    acc = acc_ref[...]
    a = jnp.concatenate([acc[0], acc[1]], axis=1) + h_ref[...]
    hb = a * (_BN_S * bg[...]) + bb[...]
    h = jnp.where(hb > 0, hb, 0.01 * hb)
    hn = _ln(h, lg[...], lb[...])
    h_out[...] = hn
    rv = jnp.dot(hn, wr[...], preferred_element_type=jnp.float32) + wbr[...]
    cv = jnp.dot(hn, wc[...], preferred_element_type=jnp.float32) + wbc[...]
    rv_out[...] = _split(rv)
    cv_out[...] = _split(cv)


def _dense_out_body(acc_ref, h_ref, bg, bb, ow, ob, p_out):
    acc = acc_ref[...]
    a = jnp.concatenate([acc[0], acc[1]], axis=1) + h_ref[...]
    hb = a * (_BN_S * bg[...]) + bb[...]
    h = jnp.where(hb > 0, hb, 0.01 * hb)
    logits = jnp.dot(h, ow[...], preferred_element_type=jnp.float32) + ob[...]
    mx = jnp.max(logits, axis=1, keepdims=True)
    ex = jnp.exp(logits - mx)
    p_out[...] = ex / jnp.sum(ex, axis=1, keepdims=True)


def _vec_spec():
    return pl.BlockSpec((1, HID), lambda i: (0, 0))


def _dense_pre(x, pre_W, pre_b, bnp_g, bnp_b, ln_g, ln_b, Wr, br, Wc, bc):
    return pl.pallas_call(
        _dense_pre_body,
        grid=(_G,),
        in_specs=[
            pl.BlockSpec((_R, C), lambda i: (i, 0)),
            pl.BlockSpec((C, HID), lambda i: (0, 0)),
            _vec_spec(), _vec_spec(), _vec_spec(), _vec_spec(), _vec_spec(),
            pl.BlockSpec((HID, HID), lambda i: (0, 0)),
            _vec_spec(),
            pl.BlockSpec((HID, HID), lambda i: (0, 0)),
            _vec_spec(),
        ],
        out_specs=[
            pl.BlockSpec((_R, HID), lambda i: (i, 0)),
            pl.BlockSpec((2, _R, FH), lambda i: (0, i, 0)),
            pl.BlockSpec((2, _R, FH), lambda i: (0, i, 0)),
        ],
        out_shape=[
            jax.ShapeDtypeStruct((N, HID), jnp.float32),
            jax.ShapeDtypeStruct((2, N, FH), jnp.float32),
            jax.ShapeDtypeStruct((2, N, FH), jnp.float32),
        ],
    )(x, pre_W, pre_b.reshape(1, -1), bnp_g.reshape(1, -1),
      bnp_b.reshape(1, -1), ln_g.reshape(1, -1), ln_b.reshape(1, -1),
      Wr, br.reshape(1, -1), Wc, bc.reshape(1, -1))


def _dense_mid(acc, h, bnl_g, bnl_b, ln_g, ln_b, Wr, br, Wc, bc):
    return pl.pallas_call(
        _dense_mid_body,
        grid=(_G,),
        in_specs=[
            pl.BlockSpec((2, _R, FH), lambda i: (0, i, 0)),
            pl.BlockSpec((_R, HID), lambda i: (i, 0)),
            _vec_spec(), _vec_spec(), _vec_spec(), _vec_spec(),
            pl.BlockSpec((HID, HID), lambda i: (0, 0)),
            _vec_spec(),
            pl.BlockSpec((HID, HID), lambda i: (0, 0)),
            _vec_spec(),
        ],
        out_specs=[
            pl.BlockSpec((_R, HID), lambda i: (i, 0)),
            pl.BlockSpec((2, _R, FH), lambda i: (0, i, 0)),
            pl.BlockSpec((2, _R, FH), lambda i: (0, i, 0)),
        ],
        out_shape=[
            jax.ShapeDtypeStruct((N, HID), jnp.float32),
            jax.ShapeDtypeStruct((2, N, FH), jnp.float32),
            jax.ShapeDtypeStruct((2, N, FH), jnp.float32),
        ],
    )(acc, h, bnl_g.reshape(1, -1), bnl_b.reshape(1, -1),
      ln_g.reshape(1, -1), ln_b.reshape(1, -1), Wr, br.reshape(1, -1),
      Wc, bc.reshape(1, -1))


def _dense_out(acc, h, bnl_g, bnl_b, out_W, out_b):
    return pl.pallas_call(
        _dense_out_body,
        grid=(_G,),
        in_specs=[
            pl.BlockSpec((2, _R, FH), lambda i: (0, i, 0)),
            pl.BlockSpec((_R, HID), lambda i: (i, 0)),
            _vec_spec(), _vec_spec(),
            pl.BlockSpec((HID, NCLS), lambda i: (0, 0)),
            pl.BlockSpec((1, NCLS), lambda i: (0, 0)),
        ],
        out_specs=[pl.BlockSpec((_R, NCLS), lambda i: (i, 0))],
        out_shape=[jax.ShapeDtypeStruct((N, NCLS), jnp.float32)],
    )(acc, h, bnl_g.reshape(1, -1), bnl_b.reshape(1, -1),
      out_W, out_b.reshape(1, -1))[0]


def _sc_body(rowv, colv, rs, rd, rvals, cs, cd, cvals, out,
             accum, idxb, srcb, valb, gbuf,
             sem_e0, sem_e1, sem_g0, sem_g1, sem_g2, sem_g3,
             sem_s0, sem_s1, sem_s2, sem_s3):
    sems_e = (sem_e0, sem_e1)
    sems_g = (sem_g0, sem_g1, sem_g2, sem_g3)
    sems_s = (sem_s0, sem_s1, sem_s2, sem_s3)
    c = lax.axis_index("c")
    s = lax.axis_index("s")
    coff = c * N
    r0 = s * RPT

    def _zb(e, _):
        z = jnp.zeros((16,), jnp.float32)
        gbuf[e, pl.ds(0, 16)] = z
        gbuf[e, pl.ds(16, 16)] = z
        return 0
    lax.fori_loop(0, BLK, _zb, 0)
    for k in range(RPT // BLK):
        pltpu.sync_copy(gbuf, accum.at[pl.ds(r0 + k * BLK, BLK)])
    rem = RPT % BLK
    if rem:
        pltpu.sync_copy(gbuf.at[pl.ds(0, rem)],
                        accum.at[pl.ds(r0 + (RPT // BLK) * BLK, rem)])
    plsc.subcore_barrier()

    def run_edges(src2, dst2, vals, feat):
        ebase2 = s * (EPT // CH)

        def fire_idx(slot, bi):
            o2 = ebase2 + bi * NCH
            pltpu.async_copy(dst2.at[pl.ds(o2, NCH)], idxb.at[slot],
                             sems_e[slot])
            pltpu.async_copy(src2.at[pl.ds(o2, NCH)], srcb.at[slot],
                             sems_e[slot])
            pltpu.async_copy(vals.at[pl.ds(s * EPT + bi * BLK, BLK)],
                             valb.at[slot], sems_e[slot])

        def wait_idx(slot):
            pltpu.make_async_copy(dst2.at[pl.ds(0, NCH)], idxb.at[slot],
                                  sems_e[slot]).wait()
            pltpu.make_async_copy(src2.at[pl.ds(0, NCH)], srcb.at[slot],
                                  sems_e[slot]).wait()
            pltpu.make_async_copy(vals.at[pl.ds(0, BLK)], valb.at[slot],
                                  sems_e[slot]).wait()

        def drain_scatters():
            for k in range(NCH):
                pltpu.make_async_copy(rowv.at[pl.ds(0, CH)],
                                      gbuf.at[pl.ds(k * CH, CH)],
                                      sems_s[k]).wait()

        def process_block(slot, bi):
            @pl.when(bi > 0)
            def _():
                drain_scatters()
            descs = []
            for k in range(NCH):
                descs.append(pltpu.async_copy(
                    feat.at[idxb.at[slot, k]],
                    gbuf.at[pl.ds(k * CH, CH)], sems_g[k]))
            for k in range(NCH):
                descs[k].wait()

                def _mul(g, _):
                    vv = valb[slot, pl.ds(k * CH + g * 16, 16)]
                    for j in range(16):
                        e = k * CH + g * 16 + j
                        v = vv[j]
                        gbuf[e, pl.ds(0, 16)] = gbuf[e, pl.ds(0, 16)] * v
                        gbuf[e, pl.ds(16, 16)] = gbuf[e, pl.ds(16, 16)] * v
                    return 0
                lax.fori_loop(0, CH // 16, _mul, 0)
                pltpu.async_copy(gbuf.at[pl.ds(k * CH, CH)],
                                 accum.at[srcb.at[slot, k]],
                                 sems_s[k], add=True)

        fire_idx(0, jnp.int32(0))

        def blk2_body(i, _):
            b0 = 2 * i
            b1 = 2 * i + 1
            fire_idx(1, b1)
            wait_idx(0)
            process_block(0, b0)
            fire_idx(0, lax.rem(b1 + 1, NBLK))
            wait_idx(1)
            process_block(1, b1)
            return 0

        lax.fori_loop(0, NBLK // 2, blk2_body, 0)
        drain_scatters()
        wait_idx(0)

    run_edges(rs, rd, rvals, rowv.at[pl.ds(coff, N)])
    run_edges(cs, cd, cvals, colv.at[pl.ds(coff, N)])
    plsc.subcore_barrier()

    for k in range(RPT // BLK):
        pltpu.sync_copy(accum.at[pl.ds(r0 + k * BLK, BLK)],
                        out.at[pl.ds(coff + r0 + k * BLK, BLK)])
    if rem:
        pltpu.sync_copy(accum.at[pl.ds(r0 + (RPT // BLK) * BLK, rem)],
                        out.at[pl.ds(coff + r0 + (RPT // BLK) * BLK, rem)])


_sc_edge = pl.kernel(
    _sc_body,
    out_type=jax.ShapeDtypeStruct((2 * N, FH), jnp.float32),
    mesh=plsc.VectorSubcoreMesh(core_axis_name="c", subcore_axis_name="s"),
    compiler_params=pltpu.CompilerParams(use_tc_tiling_on_sc=False),
    scratch_types=[
        pltpu.VMEM_SHARED((N, FH), jnp.float32),
        pltpu.VMEM((2, NCH, CH), jnp.int32),
        pltpu.VMEM((2, NCH, CH), jnp.int32),
        pltpu.VMEM((2, BLK), jnp.float32),
        pltpu.VMEM((BLK, FH), jnp.float32),
    ] + [pltpu.SemaphoreType.DMA] * 10,
)


def kernel(x, row_src, row_dst, row_val, col_src, col_dst, col_val,
           pre_W, pre_b, bnp_g, bnp_b,
           ln0_g, ln0_b, Wr0, br0, Wc0, bc0, bnl0_g, bnl0_b,
           ln1_g, ln1_b, Wr1, br1, Wc1, bc1, bnl1_g, bnl1_b,
           out_W, out_b):
    rs2 = row_src.reshape(-1, CH)
    rd2 = row_dst.reshape(-1, CH)
    cs2 = col_src.reshape(-1, CH)
    cd2 = col_dst.reshape(-1, CH)

    h0, rv0, cv0 = _dense_pre(x, pre_W, pre_b, bnp_g, bnp_b,
                              ln0_g, ln0_b, Wr0, br0, Wc0, bc0)
    acc0 = _sc_edge(rv0.reshape(2 * N, FH), cv0.reshape(2 * N, FH),
                    rs2, rd2, row_val, cs2, cd2, col_val)
    h1, rv1, cv1 = _dense_mid(acc0.reshape(2, N, FH), h0, bnl0_g, bnl0_b,
                              ln1_g, ln1_b, Wr1, br1, Wc1, bc1)
    acc1 = _sc_edge(rv1.reshape(2 * N, FH), cv1.reshape(2 * N, FH),
                    rs2, rd2, row_val, cs2, cd2, col_val)
    return _dense_out(acc1.reshape(2, N, FH), h1, bnl1_g, bnl1_b,
                      out_W, out_b)


# speedup vs baseline: 12.6918x; 1.1148x over previous
import functools
import math

import jax
import jax.numpy as jnp
from jax import lax
from jax.experimental import pallas as pl
from jax.experimental.pallas import tpu as pltpu
from jax.experimental.pallas import tpu_sc as plsc

N = 50176
C = 200
HID = 64
NCLS = 16
DEG = 16
E = N * DEG

FH = HID // 2
NSUB = 16
EPT = E // NSUB
BLK = 512
NBLK = EPT // BLK
CH = 128
NCH = BLK // CH
RPT = N // NSUB
_BN_S = 1.0 / math.sqrt(1.0 + 1e-5)
_R = 3584
_G = N // _R


def _ln(h, g, b):
    m = jnp.mean(h, axis=1, keepdims=True)
    v = jnp.mean((h - m) ** 2, axis=1, keepdims=True)
    return (h - m) * lax.rsqrt(v + 1e-5) * g + b


def _split(z):
    return jnp.stack([z[:, :FH], z[:, FH:]])


def _dense_pre_body(x_ref, pw, pb, bg, bb, lg, lb, wr, wbr, wc, wbc,
                    h_out, rv_out, cv_out):
    x = x_ref[...]
    t = jnp.dot(x, pw[...], preferred_element_type=jnp.float32) + pb[...]
    h = t * (_BN_S * bg[...]) + bb[...]
    hn = _ln(h, lg[...], lb[...])
    h_out[...] = hn
    rv = jnp.dot(hn, wr[...], preferred_element_type=jnp.float32) + wbr[...]
    cv = jnp.dot(hn, wc[...], preferred_element_type=jnp.float32) + wbc[...]
    rv_out[...] = _split(rv)
    cv_out[...] = _split(cv)


def _dense_mid_body(acc_ref, h_ref, bg, bb, lg, lb, wr, wbr, wc, wbc,
                    h_out, rv_out, cv_out):
    acc = acc_ref[...]
    a = jnp.concatenate([acc[0], acc[1]], axis=1) + h_ref[...]
    hb = a * (_BN_S * bg[...]) + bb[...]
    h = jnp.where(hb > 0, hb, 0.01 * hb)
    hn = _ln(h, lg[...], lb[...])
    h_out[...] = hn
    rv = jnp.dot(hn, wr[...], preferred_element_type=jnp.float32) + wbr[...]
    cv = jnp.dot(hn, wc[...], preferred_element_type=jnp.float32) + wbc[...]
    rv_out[...] = _split(rv)
    cv_out[...] = _split(cv)


def _dense_out_body(acc_ref, h_ref, bg, bb, ow, ob, p_out):
    acc = acc_ref[...]
    a = jnp.concatenate([acc[0], acc[1]], axis=1) + h_ref[...]
    hb = a * (_BN_S * bg[...]) + bb[...]
    h = jnp.where(hb > 0, hb, 0.01 * hb)
    logits = jnp.dot(h, ow[...], preferred_element_type=jnp.float32) + ob[...]
    mx = jnp.max(logits, axis=1, keepdims=True)
    ex = jnp.exp(logits - mx)
    p_out[...] = ex / jnp.sum(ex, axis=1, keepdims=True)


def _vec_spec():
    return pl.BlockSpec((1, HID), lambda i: (0, 0))


def _dense_pre(x, pre_W, pre_b, bnp_g, bnp_b, ln_g, ln_b, Wr, br, Wc, bc):
    return pl.pallas_call(
        _dense_pre_body,
        grid=(_G,),
        in_specs=[
            pl.BlockSpec((_R, C), lambda i: (i, 0)),
            pl.BlockSpec((C, HID), lambda i: (0, 0)),
            _vec_spec(), _vec_spec(), _vec_spec(), _vec_spec(), _vec_spec(),
            pl.BlockSpec((HID, HID), lambda i: (0, 0)),
            _vec_spec(),
            pl.BlockSpec((HID, HID), lambda i: (0, 0)),
            _vec_spec(),
        ],
        out_specs=[
            pl.BlockSpec((_R, HID), lambda i: (i, 0)),
            pl.BlockSpec((2, _R, FH), lambda i: (0, i, 0)),
            pl.BlockSpec((2, _R, FH), lambda i: (0, i, 0)),
        ],
        out_shape=[
            jax.ShapeDtypeStruct((N, HID), jnp.float32),
            jax.ShapeDtypeStruct((2, N, FH), jnp.float32),
            jax.ShapeDtypeStruct((2, N, FH), jnp.float32),
        ],
    )(x, pre_W, pre_b.reshape(1, -1), bnp_g.reshape(1, -1),
      bnp_b.reshape(1, -1), ln_g.reshape(1, -1), ln_b.reshape(1, -1),
      Wr, br.reshape(1, -1), Wc, bc.reshape(1, -1))


def _dense_mid(acc, h, bnl_g, bnl_b, ln_g, ln_b, Wr, br, Wc, bc):
    return pl.pallas_call(
        _dense_mid_body,
        grid=(_G,),
        in_specs=[
            pl.BlockSpec((2, _R, FH), lambda i: (0, i, 0)),
            pl.BlockSpec((_R, HID), lambda i: (i, 0)),
            _vec_spec(), _vec_spec(), _vec_spec(), _vec_spec(),
            pl.BlockSpec((HID, HID), lambda i: (0, 0)),
            _vec_spec(),
            pl.BlockSpec((HID, HID), lambda i: (0, 0)),
            _vec_spec(),
        ],
        out_specs=[
            pl.BlockSpec((_R, HID), lambda i: (i, 0)),
            pl.BlockSpec((2, _R, FH), lambda i: (0, i, 0)),
            pl.BlockSpec((2, _R, FH), lambda i: (0, i, 0)),
        ],
        out_shape=[
            jax.ShapeDtypeStruct((N, HID), jnp.float32),
            jax.ShapeDtypeStruct((2, N, FH), jnp.float32),
            jax.ShapeDtypeStruct((2, N, FH), jnp.float32),
        ],
    )(acc, h, bnl_g.reshape(1, -1), bnl_b.reshape(1, -1),
      ln_g.reshape(1, -1), ln_b.reshape(1, -1), Wr, br.reshape(1, -1),
      Wc, bc.reshape(1, -1))


def _dense_out(acc, h, bnl_g, bnl_b, out_W, out_b):
    return pl.pallas_call(
        _dense_out_body,
        grid=(_G,),
        in_specs=[
            pl.BlockSpec((2, _R, FH), lambda i: (0, i, 0)),
            pl.BlockSpec((_R, HID), lambda i: (i, 0)),
            _vec_spec(), _vec_spec(),
            pl.BlockSpec((HID, NCLS), lambda i: (0, 0)),
            pl.BlockSpec((1, NCLS), lambda i: (0, 0)),
        ],
        out_specs=[pl.BlockSpec((_R, NCLS), lambda i: (i, 0))],
        out_shape=[jax.ShapeDtypeStruct((N, NCLS), jnp.float32)],
    )(acc, h, bnl_g.reshape(1, -1), bnl_b.reshape(1, -1),
      out_W, out_b.reshape(1, -1))[0]


def _sc_body(rowv, colv, rs, rd, rvals, cs, cd, cvals, out,
             accum, idxb, srcb, valb, gbuf,
             sem_e0, sem_e1, sem_g0, sem_g1, sem_g2, sem_g3,
             sem_s0, sem_s1, sem_s2, sem_s3):
    sems_e = (sem_e0, sem_e1)
    sems_g = (sem_g0, sem_g1, sem_g2, sem_g3)
    sems_s = (sem_s0, sem_s1, sem_s2, sem_s3)
    c = lax.axis_index("c")
    s = lax.axis_index("s")
    coff = c * N
    r0 = s * RPT

    def _zb(e, _):
        z = jnp.zeros((16,), jnp.float32)
        gbuf[e, pl.ds(0, 16)] = z
        gbuf[e, pl.ds(16, 16)] = z
        return 0
    lax.fori_loop(0, BLK, _zb, 0)
    for k in range(RPT // BLK):
        pltpu.sync_copy(gbuf, accum.at[pl.ds(r0 + k * BLK, BLK)])
    rem = RPT % BLK
    if rem:
        pltpu.sync_copy(gbuf.at[pl.ds(0, rem)],
                        accum.at[pl.ds(r0 + (RPT // BLK) * BLK, rem)])
    plsc.subcore_barrier()

    def run_edges(src2, dst2, vals, feat):
        ebase2 = s * (EPT // CH)

        def fire_idx(slot, bi):
            o2 = ebase2 + bi * NCH
            pltpu.async_copy(dst2.at[pl.ds(o2, NCH)], idxb.at[slot],
                             sems_e[slot])
            pltpu.async_copy(src2.at[pl.ds(o2, NCH)], srcb.at[slot],
                             sems_e[slot])
            pltpu.async_copy(vals.at[pl.ds(s * EPT + bi * BLK, BLK)],
                             valb.at[slot], sems_e[slot])

        def wait_idx(slot):
            pltpu.make_async_copy(dst2.at[pl.ds(0, NCH)], idxb.at[slot],
                                  sems_e[slot]).wait()
            pltpu.make_async_copy(src2.at[pl.ds(0, NCH)], srcb.at[slot],
                                  sems_e[slot]).wait()
            pltpu.make_async_copy(vals.at[pl.ds(0, BLK)], valb.at[slot],
                                  sems_e[slot]).wait()

        def drain_scatters():
            for k in range(NCH):
                pltpu.make_async_copy(rowv.at[pl.ds(0, CH)],
                                      gbuf.at[pl.ds(k * CH, CH)],
                                      sems_s[k]).wait()

        def process_block(slot, bi):
            @pl.when(bi > 0)
            def _():
                drain_scatters()
            descs = []
            for k in range(NCH):
                descs.append(pltpu.async_copy(
                    feat.at[idxb.at[slot, k]],
                    gbuf.at[pl.ds(k * CH, CH)], sems_g[k]))
            for k in range(NCH):
                descs[k].wait()

                def _mul(g, _):
                    vv = valb[slot, pl.ds(k * CH + g * 16, 16)]
                    for j in range(16):
                        e = k * CH + g * 16 + j
                        v = vv[j]
                        gbuf[e, pl.ds(0, 16)] = gbuf[e, pl.ds(0, 16)] * v
                        gbuf[e, pl.ds(16, 16)] = gbuf[e, pl.ds(16, 16)] * v
                    return 0
                lax.fori_loop(0, CH // 16, _mul, 0)
                pltpu.async_copy(gbuf.at[pl.ds(k * CH, CH)],
                                 accum.at[srcb.at[slot, k]],
                                 sems_s[k], add=True)

        fire_idx(0, jnp.int32(0))

        def blk2_body(i, _):
            b0 = 2 * i
            b1 = 2 * i + 1
            fire_idx(1, b1)
            wait_idx(0)
            process_block(0, b0)
            fire_idx(0, lax.rem(b1 + 1, NBLK))
            wait_idx(1)
            process_block(1, b1)
            return 0

        lax.fori_loop(0, NBLK // 2, blk2_body, 0)
        drain_scatters()
        wait_idx(0)

    run_edges(rs, rd, rvals, rowv.at[pl.ds(coff, N)])
    run_edges(cs, cd, cvals, colv.at[pl.ds(coff, N)])
    plsc.subcore_barrier()

    for k in range(RPT // BLK):
        pltpu.sync_copy(accum.at[pl.ds(r0 + k * BLK, BLK)],
                        out.at[pl.ds(coff + r0 + k * BLK, BLK)])
    if rem:
        pltpu.sync_copy(accum.at[pl.ds(r0 + (RPT // BLK) * BLK, rem)],
                        out.at[pl.ds(coff + r0 + (RPT // BLK) * BLK, rem)])


_sc_edge = pl.kernel(
    _sc_body,
    out_type=jax.ShapeDtypeStruct((2 * N, FH), jnp.float32),
    mesh=plsc.VectorSubcoreMesh(core_axis_name="c", subcore_axis_name="s"),
    compiler_params=pltpu.CompilerParams(use_tc_tiling_on_sc=False),
    scratch_types=[
        pltpu.VMEM_SHARED((N, FH), jnp.float32),
        pltpu.VMEM((2, NCH, CH), jnp.int32),
        pltpu.VMEM((2, NCH, CH), jnp.int32),
        pltpu.VMEM((2, BLK), jnp.float32),
        pltpu.VMEM((BLK, FH), jnp.float32),
    ] + [pltpu.SemaphoreType.DMA] * 10,
)


def kernel(x, row_src, row_dst, row_val, col_src, col_dst, col_val,
           pre_W, pre_b, bnp_g, bnp_b,
           ln0_g, ln0_b, Wr0, br0, Wc0, bc0, bnl0_g, bnl0_b,
           ln1_g, ln1_b, Wr1, br1, Wc1, bc1, bnl1_g, bnl1_b,
           out_W, out_b):
    rs2 = row_src.reshape(-1, CH)
    rd2 = row_dst.reshape(-1, CH)
    cs2 = col_src.reshape(-1, CH)
    cd2 = col_dst.reshape(-1, CH)

    h0, rv0, cv0 = _dense_pre(x, pre_W, pre_b, bnp_g, bnp_b,
                              ln0_g, ln0_b, Wr0, br0, Wc0, bc0)
    acc0 = _sc_edge(rv0.reshape(2 * N, FH), cv0.reshape(2 * N, FH),
                    rs2, rd2, row_val, cs2, cd2, col_val)
    h1, rv1, cv1 = _dense_mid(acc0.reshape(2, N, FH), h0, bnl0_g, bnl0_b,
                              ln1_g, ln1_b, Wr1, br1, Wc1, bc1)
    acc1 = _sc_edge(rv1.reshape(2 * N, FH), cv1.reshape(2 * N, FH),
                    rs2, rd2, row_val, cs2, cd2, col_val)
    return _dense_out(acc1.reshape(2, N, FH), h1, bnl1_g, bnl1_b,
                      out_W, out_b)
